# Initial kernel scaffold; baseline (speedup 1.0000x reference)
import jax
import jax.numpy as jnp
from jax.experimental import pallas as pl


def kernel(x, anchors):
    raise NotImplementedError("write your pallas kernel here")


# speedup vs baseline: 12.2524x; 12.2524x over previous
import functools

import jax
import jax.numpy as jnp
import numpy as np
from jax import lax
from jax.experimental import pallas as pl
from jax.experimental.pallas import tpu as pltpu
from jax.experimental.pallas import tpu_sc as plsc

FH, FW, K = 48, 48, 9
N = FH * FW * K
NS = 16
PER = N // NS
VR = PER // 16
NUM_ROIS = 300
IMG = 768.0
NEG = float("-inf")
BIG = 1 << 30
ROIS_PAD = 4 * NUM_ROIS + 16


def _perm(v, idx):
    return lax.gather(
        v, idx[:, None],
        lax.GatherDimensionNumbers(offset_dims=(), collapsed_slice_dims=(0,),
                                   start_index_map=(0,)),
        slice_sizes=(1,),
        mode=lax.GatherScatterMode.PROMISE_IN_BOUNDS)


def _shuffles(iota):
    return [iota ^ s for s in (8, 4, 2, 1)]


def _allmax(v, shuf):
    for s in shuf:
        v = jnp.maximum(v, _perm(v, s))
    return v


def _allmin(v, shuf):
    for s in shuf:
        v = jnp.minimum(v, _perm(v, s))
    return v


def _nms_kernel(l0_h, l1_h, ty_h, tx_h, th_h, tw_h,
                ay1_h, ax1_h, ay2_h, ax2_h, out_h,
                l0_v, l1_v, ty_v, tx_v, th_v, tw_v,
                a1_v, a2_v, a3_v, a4_v,
                y1_v, x1_v, y2_v, x2_v, ar_v, sc_v,
                row_v, cand_v, rois_v, shared):
    cid = lax.axis_index("c")
    sid = lax.axis_index("s")
    base = sid * PER
    iota = lax.broadcasted_iota(jnp.int32, (16,), 0)
    shuf = _shuffles(iota)
    zeros_i = jnp.zeros((16,), jnp.int32)

    for src, dst in ((l0_h, l0_v), (l1_h, l1_v), (ty_h, ty_v), (tx_h, tx_v),
                     (th_h, th_v), (tw_h, tw_v), (ay1_h, a1_v), (ax1_h, a2_v),
                     (ay2_h, a3_v), (ax2_h, a4_v)):
        pltpu.sync_copy(src.at[pl.ds(base, PER)], dst)

    def zero_j(j, c):
        rois_v[pl.ds(j * 16, 16)] = jnp.zeros((16,), jnp.float32)
        return c
    lax.fori_loop(0, ROIS_PAD // 16, zero_j, 0)

    def decode_j(j, carry):
        bv, bi = carry
        sl = pl.ds(j * 16, 16)
        l0 = l0_v[sl]
        l1 = l1_v[sl]
        m = jnp.maximum(l0, l1)
        e0 = jnp.exp(l0 - m)
        e1 = jnp.exp(l1 - m)
        p = e1 / (e0 + e1)
        ay1 = a1_v[sl]
        ax1 = a2_v[sl]
        ay2 = a3_v[sl]
        ax2 = a4_v[sl]
        ah = ay2 - ay1
        aw = ax2 - ax1
        cy = ay1 + ah * 0.5 + ty_v[sl] * ah
        cx = ax1 + aw * 0.5 + tx_v[sl] * aw
        bh = ah * jnp.exp(th_v[sl])
        bw = aw * jnp.exp(tw_v[sl])
        y1 = jnp.minimum(jnp.maximum(cy - bh * 0.5, 0.0), IMG)
        x1 = jnp.minimum(jnp.maximum(cx - bw * 0.5, 0.0), IMG)
        y2 = jnp.minimum(jnp.maximum(cy + bh * 0.5, 0.0), IMG)
        x2 = jnp.minimum(jnp.maximum(cx + bw * 0.5, 0.0), IMG)
        area = jnp.maximum(y2 - y1, 0.0) * jnp.maximum(x2 - x1, 0.0)
        s = jnp.where(p >= 0.5, p, NEG)
        y1_v[sl] = y1
        x1_v[sl] = x1
        y2_v[sl] = y2
        x2_v[sl] = x2
        ar_v[sl] = area
        sc_v[sl] = s
        li = j * 16 + iota
        gt = s > bv
        return jnp.where(gt, s, bv), jnp.where(gt, li, bi)

    bv0 = jnp.full((16,), NEG, jnp.float32)
    bi0 = jnp.zeros((16,), jnp.int32)
    bv, bi = lax.fori_loop(0, VR, decode_j, (bv0, bi0))

    def publish_reduce(bv, bi):
        bm = _allmax(bv, shuf)
        bloc = _allmin(jnp.where(bv == bm, bi, BIG), shuf)
        wy1 = plsc.load_gather(y1_v, [bloc])
        wx1 = plsc.load_gather(x1_v, [bloc])
        wy2 = plsc.load_gather(y2_v, [bloc])
        wx2 = plsc.load_gather(x2_v, [bloc])
        wa = plsc.load_gather(ar_v, [bloc])
        gidxf = (bloc + base).astype(jnp.float32)
        row = jnp.where(iota == 0, bm,
              jnp.where(iota == 1, gidxf,
              jnp.where(iota == 2, wy1,
              jnp.where(iota == 3, wx1,
              jnp.where(iota == 4, wy2,
              jnp.where(iota == 5, wx2,
              jnp.where(iota == 6, wa, 0.0)))))))
        row_v[...] = row
        pltpu.sync_copy(row_v, shared.at[pl.ds(sid * 16, 16)])
        plsc.subcore_barrier()
        pltpu.sync_copy(shared, cand_v)
        tbase = iota * 16
        sc_c = plsc.load_gather(cand_v, [tbase])
        ix_c = plsc.load_gather(cand_v, [tbase + 1]).astype(jnp.int32)
        gm = _allmax(sc_c, shuf)
        gi = _allmin(jnp.where(sc_c == gm, ix_c, BIG), shuf)
        rb = (gi // PER) * 16
        gy1 = plsc.load_gather(cand_v, [rb + 2])
        gx1 = plsc.load_gather(cand_v, [rb + 3])
        gy2 = plsc.load_gather(cand_v, [rb + 4])
        gx2 = plsc.load_gather(cand_v, [rb + 5])
        ga = plsc.load_gather(cand_v, [rb + 6])
        return gm, gi, gy1, gx1, gy2, gx2, ga

    st0 = publish_reduce(bv, bi)

    def step(t, st):
        gm, gi, gy1, gx1, gy2, gx2, ga = st
        valid = gm > NEG

        @pl.when((cid == 0) & (sid == 0))
        def _store_roi():
            roi = jnp.where(iota == 0, gy1,
                  jnp.where(iota == 1, gx1,
                  jnp.where(iota == 2, gy2,
                  jnp.where(iota == 3, gx2, 0.0))))
            rois_v[pl.ds(t * 4, 16)] = jnp.where(valid, roi, 0.0)

        plsc.subcore_barrier()

        gl = gi - base

        def supp_j(j, carry):
            bv, bi, li = carry
            sl = pl.ds(j * 16, 16)
            s = sc_v[sl]
            yy1 = jnp.maximum(gy1, y1_v[sl])
            xx1 = jnp.maximum(gx1, x1_v[sl])
            yy2 = jnp.minimum(gy2, y2_v[sl])
            xx2 = jnp.minimum(gx2, x2_v[sl])
            ih = jnp.maximum(yy2 - yy1, 0.0)
            iw = jnp.maximum(xx2 - xx1, 0.0)
            inter = ih * iw
            union = ga + ar_v[sl] - inter + 1e-8
            sup = (inter > 0.5 * union) | (li == gl)
            ns = jnp.where(sup, NEG, s)
            sc_v[sl] = ns
            gt = ns > bv
            return jnp.where(gt, ns, bv), jnp.where(gt, li, bi), li + 16

        bv, bi, _ = lax.fori_loop(0, VR, supp_j, (bv0, bi0, iota))
        return publish_reduce(bv, bi)

    lax.fori_loop(0, NUM_ROIS, step, st0)

    @pl.when((cid == 0) & (sid == 0))
    def _write_out():
        pltpu.sync_copy(rois_v.at[pl.ds(0, 4 * NUM_ROIS)], out_h)


@functools.partial(
    pl.kernel,
    out_type=jax.ShapeDtypeStruct((4 * NUM_ROIS,), jnp.float32),
    mesh=plsc.VectorSubcoreMesh(core_axis_name="c", subcore_axis_name="s",
                                num_cores=1, num_subcores=16),
    compiler_params=pltpu.CompilerParams(needs_layout_passes=False),
    scratch_types=[pltpu.VMEM((PER,), jnp.float32) for _ in range(16)]
                  + [pltpu.VMEM((16,), jnp.float32),
                     pltpu.VMEM((256,), jnp.float32),
                     pltpu.VMEM((ROIS_PAD,), jnp.float32),
                     pltpu.VMEM_SHARED((256,), jnp.float32)],
)
def _nms_call(*args):
    _nms_kernel(*args)


def kernel(x, anchors):
    t = x.reshape(N, 6)
    a = anchors.reshape(N, 4)
    cols = tuple(t[:, i] for i in range(6))
    acols = tuple(a[:, i] for i in range(4))
    rois = _nms_call(*cols, *acols)
    return rois.reshape(1, NUM_ROIS, 4)
